# baseline (speedup 1.0000x reference)
import functools

import jax
import jax.numpy as jnp
from jax import lax
from jax.experimental import pallas as pl
from jax.experimental.pallas import tpu as pltpu
from jax.experimental.pallas import tpu_sc as plsc

N_BINS = 1024
BATCH = 16384

NC = 2
NS = 16
L = 16
NW = NC * NS

B_PER_W = BATCH // NW
CHUNKS = ((0, 320), (320, 192))


_mesh = plsc.VectorSubcoreMesh(core_axis_name="c", subcore_axis_name="s")


def _pipeline(table_hbm, x_hbm, y_hbm, out_hbm, xv, fv, ov, sems, base, chunks):
    s = sems[: len(chunks)]
    g = sems[len(chunks):]

    idx_cps = []
    for k, (off, csz) in enumerate(chunks):
        cx = pltpu.async_copy(
            x_hbm.at[pl.ds(base + off, csz)], xv.at[pl.ds(off, csz)], s[k]
        )
        cy = pltpu.async_copy(
            y_hbm.at[pl.ds(base + off, csz)], fv.at[pl.ds(off, csz)], s[k]
        )
        idx_cps.append((cx, cy))

    gths = []
    for k, (off, csz) in enumerate(chunks):
        cx, cy = idx_cps[k]
        cx.wait()
        cy.wait()

        @plsc.parallel_loop(off, off + csz, step=L, unroll=4)
        def idx_step(i):
            sl = pl.ds(i, L)
            x = xv[sl]
            y = fv[sl]
            fv[sl] = (
                ((x >> 3) << 13) + ((y >> 7) << 10) + ((x & 7) << 7) + (y & 127)
            )

        h = csz // 2
        gths.append(
            (
                pltpu.async_copy(
                    table_hbm.at[fv.at[pl.ds(off, h)]], ov.at[pl.ds(off, h)], g[k]
                ),
                pltpu.async_copy(
                    table_hbm.at[fv.at[pl.ds(off + h, h)]],
                    ov.at[pl.ds(off + h, h)],
                    g[k],
                ),
            )
        )

    wbs = []
    for k, (off, csz) in enumerate(chunks):
        gths[k][0].wait()
        gths[k][1].wait()
        wbs.append(
            pltpu.async_copy(
                ov.at[pl.ds(off, csz)], out_hbm.at[pl.ds(base + off, csz)], s[k]
            )
        )
    for wb in wbs:
        wb.wait()


@functools.partial(
    pl.kernel,
    mesh=_mesh,
    out_type=jax.ShapeDtypeStruct((BATCH,), jnp.float32),
    scratch_types=[
        pltpu.VMEM((B_PER_W,), jnp.int32),
        pltpu.VMEM((B_PER_W,), jnp.int32),
        pltpu.VMEM((B_PER_W,), jnp.float32),
    ]
    + [pltpu.SemaphoreType.DMA] * 4,
)
def _gather_kernel(table_hbm, x_hbm, y_hbm, out_hbm, xv, fv, ov, *sems):
    wid = lax.axis_index("s") * NC + lax.axis_index("c")
    base = wid * B_PER_W
    _pipeline(table_hbm, x_hbm, y_hbm, out_hbm, xv, fv, ov, sems, base, CHUNKS)


def kernel(energies, x_indices, y_indices):
    tiled_view = (
        energies.reshape(N_BINS // 8, 8, N_BINS // 128, 128)
        .transpose(0, 2, 1, 3)
        .reshape(N_BINS * N_BINS)
    )
    return _gather_kernel(tiled_view, x_indices, y_indices)


# speedup vs baseline: 1.0076x; 1.0076x over previous
import functools

import jax
import jax.numpy as jnp
from jax import lax
from jax.experimental import pallas as pl
from jax.experimental.pallas import tpu as pltpu
from jax.experimental.pallas import tpu_sc as plsc

N_BINS = 1024
BATCH = 16384

NC = 2
NS = 16
L = 16
NW = NC * NS

B_PER_W = BATCH // NW
CHUNKS = ((0, 320), (320, 192))


_mesh = plsc.VectorSubcoreMesh(core_axis_name="c", subcore_axis_name="s")


def _pipeline(table_hbm, x_hbm, y_hbm, out_hbm, xv, fv, ov, sems, base, chunks):
    s = sems[: len(chunks)]
    g = sems[len(chunks):]

    idx_cps = []
    for k, (off, csz) in enumerate(chunks):
        cx = pltpu.async_copy(
            x_hbm.at[pl.ds(base + off, csz)], xv.at[pl.ds(off, csz)], s[k]
        )
        cy = pltpu.async_copy(
            y_hbm.at[pl.ds(base + off, csz)], fv.at[pl.ds(off, csz)], s[k]
        )
        idx_cps.append((cx, cy))

    gths = []
    for k, (off, csz) in enumerate(chunks):
        cx, cy = idx_cps[k]
        cx.wait()
        cy.wait()

        @plsc.parallel_loop(off, off + csz, step=L, unroll=4)
        def idx_step(i):
            sl = pl.ds(i, L)
            x = xv[sl]
            y = fv[sl]
            fv[sl] = (
                ((x >> 3) << 13) + ((y >> 7) << 10) + ((x & 7) << 7) + (y & 127)
            )

        gths.append(
            pltpu.async_copy(
                table_hbm.at[fv.at[pl.ds(off, csz)]], ov.at[pl.ds(off, csz)], g[k]
            )
        )

    wbs = []
    for k, (off, csz) in enumerate(chunks):
        gths[k].wait()
        wbs.append(
            pltpu.async_copy(
                ov.at[pl.ds(off, csz)], out_hbm.at[pl.ds(base + off, csz)], s[k]
            )
        )
    for wb in wbs:
        wb.wait()


@functools.partial(
    pl.kernel,
    mesh=_mesh,
    out_type=jax.ShapeDtypeStruct((BATCH,), jnp.float32),
    scratch_types=[
        pltpu.VMEM((B_PER_W,), jnp.int32),
        pltpu.VMEM((B_PER_W,), jnp.int32),
        pltpu.VMEM((B_PER_W,), jnp.float32),
    ]
    + [pltpu.SemaphoreType.DMA] * 4,
)
def _gather_kernel(table_hbm, x_hbm, y_hbm, out_hbm, xv, fv, ov, *sems):
    wid = lax.axis_index("s") * NC + lax.axis_index("c")
    base = wid * B_PER_W
    _pipeline(table_hbm, x_hbm, y_hbm, out_hbm, xv, fv, ov, sems, base, CHUNKS)


def kernel(energies, x_indices, y_indices):
    tiled_view = (
        energies.reshape(N_BINS // 8, 8, N_BINS // 128, 128)
        .transpose(0, 2, 1, 3)
        .reshape(N_BINS * N_BINS)
    )
    return _gather_kernel(tiled_view, x_indices, y_indices)
